# Initial kernel scaffold; baseline (speedup 1.0000x reference)
import jax
import jax.numpy as jnp
from jax.experimental import pallas as pl


def kernel(x, edge_index, W_in_self, W_in_neigh, b_in, W_mid_self, W_mid_neigh, b_mid, W_out_self, W_out_neigh, b_out):
    raise NotImplementedError("write your pallas kernel here")


# speedup vs baseline: 7.8306x; 7.8306x over previous
import functools

import jax
import jax.numpy as jnp
from jax import lax
from jax.experimental import pallas as pl
from jax.experimental.pallas import tpu as pltpu
from jax.experimental.pallas import tpu_sc as plsc

N = 10000
E = 320000
H = 128
C = 3
L_MID = 12

NC = 2
NS = 16
NW = NC * NS
EPW = E // NW
K = 80
NCH = EPW // K
RPW = 624
TAIL = N - NS * RPW

_mesh = plsc.VectorSubcoreMesh(core_axis_name="c", subcore_axis_name="s")


def _copy_rows(sid, get_src, get_dst):
    pltpu.sync_copy(get_src(sid * RPW, RPW), get_dst(sid * RPW, RPW))

    @pl.when(sid == NS - 1)
    def _():
        pltpu.sync_copy(get_src(NS * RPW, TAIL), get_dst(NS * RPW, TAIL))


@functools.partial(
    pl.kernel,
    out_type=jax.ShapeDtypeStruct((NC, N, H), jnp.float32),
    mesh=_mesh,
    scratch_types=[
        pltpu.VMEM((NCH, K), jnp.int32),
        pltpu.VMEM((NCH, K), jnp.int32),
        pltpu.VMEM((K, H), jnp.float32),
        pltpu.VMEM_SHARED((N, H), jnp.float32),
        pltpu.SemaphoreType.DMA,
    ],
)
def _agg_sc(q_hbm, src3_hbm, dst3_hbm, zeros_hbm, out_hbm,
            idxs_v, idxd_v, rows_v, acc_sh, sem):
    cid = lax.axis_index("c")
    sid = lax.axis_index("s")
    wid = cid * NS + sid
    _copy_rows(sid, lambda o, n: zeros_hbm.at[pl.ds(o, n)],
               lambda o, n: acc_sh.at[pl.ds(o, n)])
    pltpu.sync_copy(src3_hbm.at[wid], idxs_v)
    pltpu.sync_copy(dst3_hbm.at[wid], idxd_v)
    plsc.subcore_barrier()

    def body(ch, carry):
        pltpu.async_copy(q_hbm.at[idxs_v.at[ch]], rows_v, sem).wait()
        pltpu.sync_copy(rows_v, acc_sh.at[idxd_v.at[ch]], add=True)
        return carry

    lax.fori_loop(0, NCH, body, 0)
    plsc.subcore_barrier()
    _copy_rows(sid, lambda o, n: acc_sh.at[pl.ds(o, n)],
               lambda o, n: out_hbm.at[cid, pl.ds(o, n)])


BN = 1000
GRID = N // BN


def _prep_body(deg_ref, x_ref, dinv_ref, q_ref):
    d = deg_ref[0, :, 0:1] + deg_ref[1, :, 0:1]
    dinv = lax.rsqrt(jnp.maximum(d, 1.0))
    dinvb = jnp.broadcast_to(dinv, (BN, H))
    dinv_ref[...] = dinvb
    q_ref[...] = x_ref[...] * dinvb


_prep_tc = pl.pallas_call(
    _prep_body,
    grid=(GRID,),
    in_specs=[
        pl.BlockSpec((NC, BN, H), lambda i: (0, i, 0)),
        pl.BlockSpec((BN, H), lambda i: (i, 0)),
    ],
    out_specs=[
        pl.BlockSpec((BN, H), lambda i: (i, 0)),
        pl.BlockSpec((BN, H), lambda i: (i, 0)),
    ],
    out_shape=[
        jax.ShapeDtypeStruct((N, H), jnp.float32),
        jax.ShapeDtypeStruct((N, H), jnp.float32),
    ],
)


def _layer_body(h_ref, s_ref, dinv_ref, ws_ref, wn_ref, b_ref,
                h_out_ref, q_out_ref, *, act):
    sb = s_ref[0] + s_ref[1]
    t = jnp.dot(sb, wn_ref[...], preferred_element_type=jnp.float32)
    z = (jnp.dot(h_ref[...], ws_ref[...], preferred_element_type=jnp.float32)
         + dinv_ref[...] * t + b_ref[...])
    if act:
        z = jnp.maximum(z, 0.0)
    h_out_ref[...] = z
    if q_out_ref is not None:
        q_out_ref[...] = z * dinv_ref[...]


def _make_layer_tc(act, with_q):
    if with_q:
        body = functools.partial(_layer_body, act=act)
        out_specs = [pl.BlockSpec((BN, H), lambda i: (i, 0)),
                     pl.BlockSpec((BN, H), lambda i: (i, 0))]
        out_shape = [jax.ShapeDtypeStruct((N, H), jnp.float32),
                     jax.ShapeDtypeStruct((N, H), jnp.float32)]
    else:
        def body(h_ref, s_ref, dinv_ref, ws_ref, wn_ref, b_ref, h_out_ref):
            _layer_body(h_ref, s_ref, dinv_ref, ws_ref, wn_ref, b_ref,
                        h_out_ref, None, act=act)
        out_specs = pl.BlockSpec((BN, H), lambda i: (i, 0))
        out_shape = jax.ShapeDtypeStruct((N, H), jnp.float32)
    return pl.pallas_call(
        body,
        grid=(GRID,),
        in_specs=[
            pl.BlockSpec((BN, H), lambda i: (i, 0)),
            pl.BlockSpec((NC, BN, H), lambda i: (0, i, 0)),
            pl.BlockSpec((BN, H), lambda i: (i, 0)),
            pl.BlockSpec((H, H), lambda i: (0, 0)),
            pl.BlockSpec((H, H), lambda i: (0, 0)),
            pl.BlockSpec((1, H), lambda i: (0, 0)),
        ],
        out_specs=out_specs,
        out_shape=out_shape,
    )


_layer_tc = _make_layer_tc(act=True, with_q=True)
_final_tc = _make_layer_tc(act=False, with_q=False)


def kernel(x, edge_index, W_in_self, W_in_neigh, b_in,
           W_mid_self, W_mid_neigh, b_mid,
           W_out_self, W_out_neigh, b_out):
    src3 = edge_index[0].astype(jnp.int32).reshape(NW, NCH, K)
    dst3 = edge_index[1].astype(jnp.int32).reshape(NW, NCH, K)
    onesNH = jnp.ones((N, H), jnp.float32)
    zerosH = jnp.zeros((N, H), jnp.float32)

    deg2 = _agg_sc(onesNH, src3, dst3, zerosH)
    dinvb, q = _prep_tc(deg2, x)

    Wso = jnp.zeros((H, H), jnp.float32).at[:, :C].set(W_out_self)
    Wno = jnp.zeros((H, H), jnp.float32).at[:, :C].set(W_out_neigh)
    bo = jnp.zeros((1, H), jnp.float32).at[0, :C].set(b_out)

    h = x
    for li in range(L_MID + 2):
        s = _agg_sc(q, src3, dst3, zerosH)
        if li == 0:
            h, q = _layer_tc(h, s, dinvb, W_in_self, W_in_neigh,
                             b_in.reshape(1, H))
        elif li <= L_MID:
            h, q = _layer_tc(h, s, dinvb, W_mid_self[li - 1],
                             W_mid_neigh[li - 1], b_mid[li - 1].reshape(1, H))
        else:
            h = _final_tc(h, s, dinvb, Wso, Wno, bo)
    return h[:, :C]
